# Initial kernel scaffold; baseline (speedup 1.0000x reference)
import jax
import jax.numpy as jnp
from jax.experimental import pallas as pl


def kernel(superpixels_results, feats):
    raise NotImplementedError("write your pallas kernel here")


# speedup vs baseline: 3.1026x; 3.1026x over previous
import functools

import jax
import jax.numpy as jnp
from jax import lax
from jax.experimental import pallas as pl
from jax.experimental.pallas import tpu as pltpu
from jax.experimental.pallas import tpu_sc as plsc

_NC = 2
_NS = 16
_NW = _NC * _NS
_LANES = 16

_C = 96
_N = 512 * 512
_K = 512
_CHUNK = 8192


def _make_sc_call(C, N, K, chunk, interpret=False):
    cpw = C // _NW
    nchunk = N // chunk
    acc_words = K * _LANES

    mesh = plsc.VectorSubcoreMesh(core_axis_name="c", subcore_axis_name="s",
                                  num_cores=_NC, num_subcores=_NS)

    @functools.partial(
        pl.kernel,
        out_type=[
            jax.ShapeDtypeStruct((C * K,), jnp.float32),
            jax.ShapeDtypeStruct((C * K,), jnp.float32),
            jax.ShapeDtypeStruct((K,), jnp.float32),
        ],
        mesh=mesh,
        scratch_types=[
            pltpu.VMEM((2 * chunk,), jnp.int32),
            pltpu.VMEM((2 * cpw * chunk,), jnp.float32),
            pltpu.VMEM((cpw * acc_words,), jnp.float32),
            pltpu.VMEM((cpw * acc_words,), jnp.float32),
            pltpu.VMEM((acc_words,), jnp.float32),
            pltpu.VMEM((cpw * K,), jnp.float32),
            pltpu.SemaphoreType.DMA,
            pltpu.SemaphoreType.DMA,
        ],
        compiler_params=pltpu.CompilerParams(needs_layout_passes=False),
        interpret=interpret,
    )
    def sc_call(x_hbm, seg_hbm, sums_hbm, sq_hbm, cnt_hbm,
                seg_v, x_v, sum_acc, sq_acc, cnt_acc, out_v, sem0, sem1):
        w = lax.axis_index("c") * _NS + lax.axis_index("s")
        c0 = w * cpw
        zeros = jnp.zeros((_LANES,), jnp.float32)
        ones = jnp.ones((_LANES,), jnp.float32)
        lane_base = lax.iota(jnp.int32, _LANES) * K

        def _copies(g, b, sem):
            off = g * chunk
            cps = [pltpu.make_async_copy(
                seg_hbm.at[pl.ds(off, chunk)],
                seg_v.at[pl.ds(b * chunk, chunk)], sem)]
            for c in range(cpw):
                cps.append(pltpu.make_async_copy(
                    x_hbm.at[pl.ds((c0 + c) * N + off, chunk)],
                    x_v.at[pl.ds((b * cpw + c) * chunk, chunk)], sem))
            return cps

        def _issue(g, b, sem):
            for cp in _copies(g, b, sem):
                cp.start()

        def _wait(g, b, sem):
            for cp in _copies(g, b, sem):
                cp.wait()

        _issue(0, 0, sem0)

        def _zero(i, _):
            for c in range(cpw):
                sl = pl.ds(c * acc_words + i * _LANES, _LANES)
                sum_acc[sl] = zeros
                sq_acc[sl] = zeros
            cnt_acc[pl.ds(i * _LANES, _LANES)] = zeros
            return 0
        lax.fori_loop(0, acc_words // _LANES, _zero, 0)

        unroll = 2

        def _compute(b):
            def _vec(p, _):
                for u in range(unroll):
                    base = p * (unroll * _LANES) + u * _LANES
                    idx = lane_base + seg_v[pl.ds(b * chunk + base, _LANES)]
                    for c in range(cpw):
                        xv = x_v[pl.ds((b * cpw + c) * chunk + base, _LANES)]
                        idx_c = idx + (c * acc_words)
                        plsc.addupdate_scatter(sum_acc, [idx_c], xv)
                        plsc.addupdate_scatter(sq_acc, [idx_c], xv * xv)

                    @pl.when(w == 0)
                    def _():
                        plsc.addupdate_scatter(cnt_acc, [idx], ones)
                return 0
            lax.fori_loop(0, chunk // (unroll * _LANES), _vec, 0)

        def _pair(gg, _):
            g0 = gg * 2
            _issue(g0 + 1, 1, sem1)
            _wait(g0, 0, sem0)
            _compute(0)

            @pl.when(g0 + 2 < nchunk)
            def _():
                _issue(g0 + 2, 0, sem0)
            _wait(g0 + 1, 1, sem1)
            _compute(1)
            return 0
        lax.fori_loop(0, nchunk // 2, _pair, 0)

        def _reduce_rows(acc_ref):
            def _col(j, _):
                base = j * _LANES
                for c in range(cpw):
                    v = acc_ref[pl.ds(c * acc_words + base, _LANES)]
                    for r in range(1, _LANES):
                        v = v + acc_ref[pl.ds(c * acc_words + r * K + base,
                                              _LANES)]
                    out_v[pl.ds(c * K + base, _LANES)] = v
                return 0
            lax.fori_loop(0, K // _LANES, _col, 0)

        _reduce_rows(sum_acc)
        pltpu.sync_copy(out_v, sums_hbm.at[pl.ds(c0 * K, cpw * K)])
        _reduce_rows(sq_acc)
        pltpu.sync_copy(out_v, sq_hbm.at[pl.ds(c0 * K, cpw * K)])

        @pl.when(w == 0)
        def _():
            def _colc(j, _):
                base = j * _LANES
                v = cnt_acc[pl.ds(base, _LANES)]
                for r in range(1, _LANES):
                    v = v + cnt_acc[pl.ds(r * K + base, _LANES)]
                out_v[pl.ds(base, _LANES)] = v
                return 0
            lax.fori_loop(0, K // _LANES, _colc, 0)
            pltpu.sync_copy(out_v.at[pl.ds(0, K)], cnt_hbm)

    return sc_call


def _epilogue_body(cnt_ref, sums_ref, sq_ref, out_ref):
    counts = cnt_ref[...]
    safe = jnp.maximum(counts, 1.0)
    s = sums_ref[...]
    q = sq_ref[...]
    means = s / safe
    var_sum = jnp.sum(q - safe * means * means, axis=0, keepdims=True)
    per_seg = var_sum / (sums_ref.shape[0] * safe)
    var_loss = jnp.sum(jnp.where(counts >= 2.0, per_seg,
                                 jnp.zeros_like(per_seg)))
    c = jnp.sum((counts > 0.0).astype(jnp.float32))
    out_ref[0, 0] = var_loss / c


def _epilogue(counts, sums, sq):
    return pl.pallas_call(
        _epilogue_body,
        out_shape=jax.ShapeDtypeStruct((1, 1), jnp.float32),
        out_specs=pl.BlockSpec(memory_space=pltpu.SMEM),
    )(counts, sums, sq)


_SC_CALL = _make_sc_call(_C, _N, _K, _CHUNK)


def kernel(superpixels_results, feats):
    seg = superpixels_results.reshape(-1)
    x = feats.reshape(-1)
    sums, sq, cnt = _SC_CALL(x, seg)
    loss = _epilogue(cnt.reshape(1, _K),
                     sums.reshape(_C, _K), sq.reshape(_C, _K))
    return loss[0, 0]


# speedup vs baseline: 5.0720x; 1.6348x over previous
import functools

import jax
import jax.numpy as jnp
from jax import lax
from jax.experimental import pallas as pl
from jax.experimental.pallas import tpu as pltpu
from jax.experimental.pallas import tpu_sc as plsc

_NC = 2
_NS = 16
_NW = _NC * _NS
_LANES = 16

_C = 96
_H = 512
_W = 512
_K = 512
_ROWS = 16


def _make_sc_call(C, H, W, K, rows, interpret=False):
    cpw = C // _NW
    chunk = rows * W
    nchunk = H // rows
    acc_words = K * _LANES

    mesh = plsc.VectorSubcoreMesh(core_axis_name="c", subcore_axis_name="s",
                                  num_cores=_NC, num_subcores=_NS)

    @functools.partial(
        pl.kernel,
        out_type=[
            jax.ShapeDtypeStruct((C * acc_words,), jnp.float32),
            jax.ShapeDtypeStruct((C * acc_words,), jnp.float32),
            jax.ShapeDtypeStruct((acc_words,), jnp.float32),
        ],
        mesh=mesh,
        scratch_types=[
            pltpu.VMEM((2, rows, W), jnp.int32),
            pltpu.VMEM((2, cpw, rows, W), jnp.float32),
            pltpu.VMEM((cpw * acc_words,), jnp.float32),
            pltpu.VMEM((cpw * acc_words,), jnp.float32),
            pltpu.VMEM((acc_words,), jnp.float32),
            pltpu.SemaphoreType.DMA,
            pltpu.SemaphoreType.DMA,
        ],
        compiler_params=pltpu.CompilerParams(needs_layout_passes=False),
        interpret=interpret,
    )
    def sc_call(x_hbm, seg_hbm, sums_hbm, sq_hbm, cnt_hbm,
                seg_v, x_v, sum_acc, sq_acc, cnt_acc, sem0, sem1):
        w = lax.axis_index("c") * _NS + lax.axis_index("s")
        c0 = w * cpw
        zeros = jnp.zeros((_LANES,), jnp.float32)
        ones = jnp.ones((_LANES,), jnp.float32)
        lane = lax.iota(jnp.int32, _LANES)

        def _copies(g, b, sem):
            r0 = g * rows
            return [
                pltpu.make_async_copy(
                    seg_hbm.at[0, pl.ds(r0, rows), :], seg_v.at[b], sem),
                pltpu.make_async_copy(
                    x_hbm.at[0, pl.ds(c0, cpw), pl.ds(r0, rows), :],
                    x_v.at[b], sem),
            ]

        def _issue(g, b, sem):
            for cp in _copies(g, b, sem):
                cp.start()

        def _wait(g, b, sem):
            for cp in _copies(g, b, sem):
                cp.wait()

        _issue(0, 0, sem0)

        def _zero(i, _):
            for c in range(cpw):
                sum_acc[pl.ds(c * acc_words + i * _LANES, _LANES)] = zeros
                sq_acc[pl.ds(c * acc_words + i * _LANES, _LANES)] = zeros
            cnt_acc[pl.ds(i * _LANES, _LANES)] = zeros
            return 0
        lax.fori_loop(0, acc_words // _LANES, _zero, 0)

        unroll = 4
        vecs_per_row = W // _LANES

        def _compute(b):
            def _vec(p, _):
                for u in range(unroll):
                    q = p * unroll + u
                    rr = q // vecs_per_row
                    col = (q % vecs_per_row) * _LANES
                    seg = seg_v[b, rr, pl.ds(col, _LANES)]
                    idx = seg * _LANES + lane
                    for c in range(cpw):
                        xv = x_v[b, c, rr, pl.ds(col, _LANES)]
                        idx_c = idx + (c * acc_words)
                        plsc.addupdate_scatter(sum_acc, [idx_c], xv)
                        plsc.addupdate_scatter(sq_acc, [idx_c], xv * xv)

                    @pl.when(w == 0)
                    def _():
                        plsc.addupdate_scatter(cnt_acc, [idx], ones)
                return 0
            lax.fori_loop(0, chunk // (unroll * _LANES), _vec, 0)

        def _pair(gg, _):
            g0 = gg * 2
            _issue(g0 + 1, 1, sem1)
            _wait(g0, 0, sem0)
            _compute(0)

            @pl.when(g0 + 2 < nchunk)
            def _():
                _issue(g0 + 2, 0, sem0)
            _wait(g0 + 1, 1, sem1)
            _compute(1)
            return 0
        lax.fori_loop(0, nchunk // 2, _pair, 0)

        pltpu.sync_copy(sum_acc,
                        sums_hbm.at[pl.ds(c0 * acc_words, cpw * acc_words)])
        pltpu.sync_copy(sq_acc,
                        sq_hbm.at[pl.ds(c0 * acc_words, cpw * acc_words)])

        @pl.when(w == 0)
        def _():
            pltpu.sync_copy(cnt_acc, cnt_hbm)

    return sc_call


def _epilogue_body(cnt_ref, sums_ref, sq_ref, out_ref):
    C = sums_ref.shape[0] // _LANES
    i = lax.broadcasted_iota(jnp.int32, (_W, 32), 0)
    j = lax.broadcasted_iota(jnp.int32, (_W, 32), 1)
    fold = (i // _LANES == j).astype(jnp.float32)
    cnt = jnp.dot(cnt_ref[...], fold,
                  preferred_element_type=jnp.float32)
    s = jnp.dot(sums_ref[...], fold,
                preferred_element_type=jnp.float32)
    q = jnp.dot(sq_ref[...], fold,
                preferred_element_type=jnp.float32)
    safe = jnp.maximum(cnt, 1.0)
    s3 = s.reshape(C, _LANES, 32)
    q3 = q.reshape(C, _LANES, 32)
    means = s3 / safe[None]
    var_sum = jnp.sum(q3 - safe[None] * means * means, axis=0)
    per_seg = var_sum / (C * safe)
    var_loss = jnp.sum(jnp.where(cnt >= 2.0, per_seg,
                                 jnp.zeros_like(per_seg)))
    c = jnp.sum((cnt > 0.0).astype(jnp.float32))
    out_ref[0, 0] = var_loss / c


def _epilogue(counts, sums, sq):
    return pl.pallas_call(
        _epilogue_body,
        out_shape=jax.ShapeDtypeStruct((1, 1), jnp.float32),
        out_specs=pl.BlockSpec(memory_space=pltpu.SMEM),
    )(counts, sums, sq)


_SC_CALL = _make_sc_call(_C, _H, _W, _K, _ROWS)


def kernel(superpixels_results, feats):
    sums, sq, cnt = _SC_CALL(feats, superpixels_results)
    loss = _epilogue(cnt.reshape(_LANES, _W),
                     sums.reshape(_C * _LANES, _W),
                     sq.reshape(_C * _LANES, _W))
    return loss[0, 0]


# speedup vs baseline: 11.2299x; 2.2141x over previous
import functools

import jax
import jax.numpy as jnp
from jax import lax
from jax.experimental import pallas as pl
from jax.experimental.pallas import tpu as pltpu
from jax.experimental.pallas import tpu_sc as plsc

_NC = 2
_NS = 16
_NW = _NC * _NS
_LANES = 16

_C = 96
_H = 512
_W = 512
_K = 512
_ROWS = 16


def _make_sc_call(C, H, W, K, rows, interpret=False):
    cpw = C // _NW
    chunk = rows * W
    nchunk = H // rows
    acc_words = K * _LANES

    mesh = plsc.VectorSubcoreMesh(core_axis_name="c", subcore_axis_name="s",
                                  num_cores=_NC, num_subcores=_NS)

    @functools.partial(
        pl.kernel,
        out_type=[
            jax.ShapeDtypeStruct((C * acc_words,), jnp.float32),
            jax.ShapeDtypeStruct((C * acc_words,), jnp.float32),
            jax.ShapeDtypeStruct((acc_words,), jnp.float32),
        ],
        mesh=mesh,
        scratch_types=[
            pltpu.VMEM((2, rows, W), jnp.int32),
            pltpu.VMEM((2, cpw, rows, W), jnp.float32),
            pltpu.VMEM((cpw * acc_words,), jnp.float32),
            pltpu.VMEM((cpw * acc_words,), jnp.float32),
            pltpu.VMEM((acc_words,), jnp.float32),
            pltpu.SemaphoreType.DMA,
            pltpu.SemaphoreType.DMA,
        ],
        compiler_params=pltpu.CompilerParams(needs_layout_passes=False),
        interpret=interpret,
    )
    def sc_call(x_hbm, seg_hbm, sums_hbm, sq_hbm, cnt_hbm,
                seg_v, x_v, sum_acc, sq_acc, cnt_acc, sem0, sem1):
        w = lax.axis_index("c") * _NS + lax.axis_index("s")
        c0 = w * cpw
        zeros = jnp.zeros((_LANES,), jnp.float32)
        ones = jnp.ones((_LANES,), jnp.float32)
        lane = lax.iota(jnp.int32, _LANES)

        def _copies(g, b, sem):
            r0 = g * rows
            return [
                pltpu.make_async_copy(
                    seg_hbm.at[0, pl.ds(r0, rows), :], seg_v.at[b], sem),
                pltpu.make_async_copy(
                    x_hbm.at[0, pl.ds(c0, cpw), pl.ds(r0, rows), :],
                    x_v.at[b], sem),
            ]

        def _issue(g, b, sem):
            for cp in _copies(g, b, sem):
                cp.start()

        def _wait(g, b, sem):
            for cp in _copies(g, b, sem):
                cp.wait()

        _issue(0, 0, sem0)

        def _zero(i, _):
            for c in range(cpw):
                sum_acc[pl.ds(c * acc_words + i * _LANES, _LANES)] = zeros
                sq_acc[pl.ds(c * acc_words + i * _LANES, _LANES)] = zeros
            cnt_acc[pl.ds(i * _LANES, _LANES)] = zeros
            return 0
        lax.fori_loop(0, acc_words // _LANES, _zero, 0)

        unroll = 4

        def _compute(b):
            def _row(rr, _):
                def _blk(jb, _):
                    idxs = []
                    vals = []
                    for u in range(unroll):
                        col = (jb * unroll + u) * _LANES
                        seg = seg_v[b, rr, pl.ds(col, _LANES)]
                        idxs.append(seg * _LANES + lane)
                        vals.append([x_v[b, c, rr, pl.ds(col, _LANES)]
                                     for c in range(cpw)])
                    for u in range(unroll):
                        idx = idxs[u]
                        for c in range(cpw):
                            xv = vals[u][c]
                            idx_c = idx + (c * acc_words)
                            plsc.addupdate_scatter(sum_acc, [idx_c], xv)
                            plsc.addupdate_scatter(sq_acc, [idx_c], xv * xv)

                    @pl.when(w == 0)
                    def _():
                        for u in range(unroll):
                            plsc.addupdate_scatter(cnt_acc, [idxs[u]], ones)
                    return 0
                lax.fori_loop(0, W // (unroll * _LANES), _blk, 0)
                return 0
            lax.fori_loop(0, rows, _row, 0)

        def _pair(gg, _):
            g0 = gg * 2
            _issue(g0 + 1, 1, sem1)
            _wait(g0, 0, sem0)
            _compute(0)

            @pl.when(g0 + 2 < nchunk)
            def _():
                _issue(g0 + 2, 0, sem0)
            _wait(g0 + 1, 1, sem1)
            _compute(1)
            return 0
        lax.fori_loop(0, nchunk // 2, _pair, 0)

        pltpu.sync_copy(sum_acc,
                        sums_hbm.at[pl.ds(c0 * acc_words, cpw * acc_words)])
        pltpu.sync_copy(sq_acc,
                        sq_hbm.at[pl.ds(c0 * acc_words, cpw * acc_words)])

        @pl.when(w == 0)
        def _():
            pltpu.sync_copy(cnt_acc, cnt_hbm)

    return sc_call


def _epilogue_body(cnt_ref, sums_ref, sq_ref, out_ref):
    C = sums_ref.shape[0] // _LANES
    i = lax.broadcasted_iota(jnp.int32, (_W, 32), 0)
    j = lax.broadcasted_iota(jnp.int32, (_W, 32), 1)
    fold = (i // _LANES == j).astype(jnp.float32)
    cnt = jnp.dot(cnt_ref[...], fold,
                  preferred_element_type=jnp.float32)
    s = jnp.dot(sums_ref[...], fold,
                preferred_element_type=jnp.float32)
    q = jnp.dot(sq_ref[...], fold,
                preferred_element_type=jnp.float32)
    safe = jnp.maximum(cnt, 1.0)
    s3 = s.reshape(C, _LANES, 32)
    q3 = q.reshape(C, _LANES, 32)
    means = s3 / safe[None]
    var_sum = jnp.sum(q3 - safe[None] * means * means, axis=0)
    per_seg = var_sum / (C * safe)
    var_loss = jnp.sum(jnp.where(cnt >= 2.0, per_seg,
                                 jnp.zeros_like(per_seg)))
    c = jnp.sum((cnt > 0.0).astype(jnp.float32))
    out_ref[0, 0] = var_loss / c


def _epilogue(counts, sums, sq):
    return pl.pallas_call(
        _epilogue_body,
        out_shape=jax.ShapeDtypeStruct((1, 1), jnp.float32),
        out_specs=pl.BlockSpec(memory_space=pltpu.SMEM),
    )(counts, sums, sq)


_SC_CALL = _make_sc_call(_C, _H, _W, _K, _ROWS)


def kernel(superpixels_results, feats):
    sums, sq, cnt = _SC_CALL(feats, superpixels_results)
    loss = _epilogue(cnt.reshape(_LANES, _W),
                     sums.reshape(_C * _LANES, _W),
                     sq.reshape(_C * _LANES, _W))
    return loss[0, 0]
